# baseline (speedup 1.0000x reference)
import math

import jax
import jax.numpy as jnp
from jax.experimental import pallas as pl
from jax.experimental.pallas import tpu as pltpu

_BLK = 512


def _vq_body(p_ref, tab_ref, logit_ref, lpmf_ref, pq_ref, bit_ref,
             tab2_ref, tnorm_ref, gl_ref):
    i = pl.program_id(0)
    nblk = pl.num_programs(0)
    d = tab_ref.shape[0]
    c = tab_ref.shape[1]

    @pl.when(i == 0)
    def _precompute():
        tab = tab_ref[...]
        tab2_ref[...] = tab + tab
        tnorm_ref[...] = jnp.sum(tab * tab, axis=1)[None, :]
        mx = jnp.max(tab, axis=1, keepdims=True)
        ex = jnp.exp(tab - mx)
        ls = tab - (jnp.log(jnp.sum(ex, axis=1, keepdims=True)) + mx)
        lg = logit_ref[...]
        ml = jnp.max(lg)
        llc = lg - (jnp.log(jnp.sum(jnp.exp(lg - ml))) + ml)
        llc_col = (llc * (-1.0 / math.log(2.0))).reshape(d, 1)
        gl_ref[:, :c] = tab.astype(jnp.bfloat16)
        gl_ref[:, c:2 * c] = ls.astype(jnp.bfloat16)
        gl_ref[:, 2 * c:2 * c + 1] = llc_col.astype(jnp.bfloat16)
        gl_ref[:, 2 * c + 1:] = jnp.zeros((d, 7), jnp.bfloat16)

    p = jnp.clip(p_ref[...], -1.0, 1.0)
    scores = jax.lax.dot_general(
        p, tab2_ref[...], (((1,), (1,)), ((), ())),
        preferred_element_type=jnp.float32)
    neg = scores - tnorm_ref[...]
    idx = jnp.argmax(neg, axis=1)
    iota = jax.lax.broadcasted_iota(jnp.int32, neg.shape, 1)
    oh = (iota == idx[:, None]).astype(jnp.float32).astype(jnp.bfloat16)

    qg = jax.lax.dot_general(
        oh, gl_ref[...], (((1,), (0,)), ((), ())),
        preferred_element_type=jnp.float32)
    pq_ref[...] = qg[:, :c]
    lpmf_ref[...] = qg[:, c:2 * c]
    bit_blk = jnp.sum(qg[:, 2 * c:])

    @pl.when(i == 0)
    def _():
        bit_ref[0, 0] = bit_blk

    @pl.when(i > 0)
    def _():
        bit_ref[0, 0] += bit_blk


def kernel(params, param_table, logits):
    a, b, c = params.shape
    d = param_table.shape[0]
    tokens = a * b
    p2 = params.reshape(tokens, c)
    lg2 = logits.reshape(1, d)
    grid = tokens // _BLK

    lpmf, pq, bit = pl.pallas_call(
        _vq_body,
        grid=(grid,),
        in_specs=[
            pl.BlockSpec((_BLK, c), lambda i: (i, 0)),
            pl.BlockSpec((d, c), lambda i: (0, 0)),
            pl.BlockSpec((1, d), lambda i: (0, 0)),
        ],
        out_specs=[
            pl.BlockSpec((_BLK, c), lambda i: (i, 0)),
            pl.BlockSpec((_BLK, c), lambda i: (i, 0)),
            pl.BlockSpec(memory_space=pltpu.SMEM),
        ],
        out_shape=[
            jax.ShapeDtypeStruct((tokens, c), jnp.float32),
            jax.ShapeDtypeStruct((tokens, c), jnp.float32),
            jax.ShapeDtypeStruct((1, 1), jnp.float32),
        ],
        scratch_shapes=[
            pltpu.VMEM((d, c), jnp.float32),
            pltpu.VMEM((1, d), jnp.float32),
            pltpu.VMEM((d, 2 * c + 8), jnp.bfloat16),
        ],
        compiler_params=pltpu.CompilerParams(
            dimension_semantics=("arbitrary",),
        ),
    )(p2, param_table, lg2)

    return (lpmf.reshape(a, b, c), pq.reshape(a, b, c), bit[0, 0])
